# baseline (speedup 1.0000x reference)
import functools

import jax
import jax.numpy as jnp
from jax import lax
from jax.experimental import pallas as pl
from jax.experimental.pallas import tpu as pltpu, tpu_sc as plsc

N = 10000
E = 320000
D = 128
H = 512
E2 = 2 * E

_NC, _NS = 2, 16
_NW = _NC * _NS
_CH = 128
_NCHUNK = 160
_EPT = _CH * _NCHUNK
_EP = _NW * _EPT
_EPR = _EP // 128
_NT = 10240
_ZR = _NT // _NS
_IB = 32
_NG = _NCHUNK // _IB

_RB = 1000
_GRID = N // _RB


def _rng_consts():
    ka, kb = jax.random.split(jax.random.key(42))
    out = []
    for kk in (ka, kb):
        k1, k2 = jax.random.split(kk)
        keep = jax.random.bernoulli(k1, 0.8, (E2,))
        perm = jax.random.permutation(k2, N)
        mvec = jnp.zeros((N,), dtype=bool).at[perm[: N // 2]].set(True)
        out.append((keep.astype(jnp.int32), mvec.astype(jnp.float32)))
    return out


def _pad_edges(v):
    return jnp.concatenate(
        [v, jnp.zeros((_EP - E2,), v.dtype)]).reshape(_EPR, 128)


def _sc_agg_body(src_ref, dst_ref, tbl_ref, out_ref,
                 idx_s, idx_d, buf0, buf1, agg, gsem0, gsem1):
    c = lax.axis_index("c")
    s = lax.axis_index("s")
    w = s * _NC + c

    zv = jnp.zeros((16,), jnp.float32)

    def zrow(i, carry):
        for j in range(8):
            buf0[i, pl.ds(j * 16, 16)] = zv
        return carry

    lax.fori_loop(0, _CH, zrow, 0)
    for t in range(_ZR // _CH):
        pltpu.sync_copy(buf0, agg.at[pl.ds(s * _ZR + t * _CH, _CH)])
    plsc.subcore_barrier()

    def group(g, carry):
        base_row = w * _NCHUNK + g * _IB
        pltpu.sync_copy(src_ref.at[pl.ds(base_row, _IB)], idx_s)
        pltpu.sync_copy(dst_ref.at[pl.ds(base_row, _IB)], idx_d)
        pltpu.async_copy(tbl_ref.at[idx_s.at[0]], buf0, gsem0)
        pltpu.async_copy(tbl_ref.at[idx_s.at[1]], buf1, gsem1)

        def step(i, cc):
            j0 = 2 * i
            pltpu.make_async_copy(tbl_ref.at[idx_s.at[j0]], buf0, gsem0).wait()
            pltpu.sync_copy(buf0, agg.at[idx_d.at[j0]], add=True)

            @pl.when(j0 + 2 < _IB)
            def _():
                pltpu.async_copy(tbl_ref.at[idx_s.at[j0 + 2]], buf0, gsem0)

            j1 = j0 + 1
            pltpu.make_async_copy(tbl_ref.at[idx_s.at[j1]], buf1, gsem1).wait()
            pltpu.sync_copy(buf1, agg.at[idx_d.at[j1]], add=True)

            @pl.when(j1 + 2 < _IB)
            def _():
                pltpu.async_copy(tbl_ref.at[idx_s.at[j1 + 2]], buf1, gsem1)

            return cc

        lax.fori_loop(0, _IB // 2, step, 0)
        return carry

    lax.fori_loop(0, _NG, group, 0)
    plsc.subcore_barrier()

    for t in range(_ZR // _CH):
        r0 = s * _ZR + t * _CH
        pltpu.sync_copy(agg.at[pl.ds(r0, _CH)], buf0)
        pltpu.sync_copy(buf0, out_ref.at[c, pl.ds(r0, _CH)])


@functools.cache
def _sc_agg_call():
    return pl.kernel(
        _sc_agg_body,
        out_type=jax.ShapeDtypeStruct((_NC, _NT, D), jnp.float32),
        mesh=plsc.VectorSubcoreMesh(core_axis_name="c", subcore_axis_name="s",
                                    num_cores=_NC, num_subcores=_NS),
        scratch_types=[
            pltpu.VMEM((_IB, _CH), jnp.int32),
            pltpu.VMEM((_IB, _CH), jnp.int32),
            pltpu.VMEM((_CH, D), jnp.float32),
            pltpu.VMEM((_CH, D), jnp.float32),
            pltpu.VMEM_SHARED((_NT, D), jnp.float32),
            pltpu.SemaphoreType.DMA,
            pltpu.SemaphoreType.DMA,
        ],
    )


def _sc_agg(srcp, dstp, table):
    return _sc_agg_call()(srcp, dstp, table)


def _prep_body(d_ref, b_ref, kb_ref, o1_ref, o2_ref):
    d = d_ref[...]
    b = b_ref[...]
    kb = kb_ref[...]
    trash = N + lax.broadcasted_iota(jnp.int32, d.shape, 1)
    o1_ref[...] = jnp.where((b & kb & 1) != 0, d, trash)
    o2_ref[...] = jnp.where((b & (kb >> 1) & 1) != 0, d, trash)


def _prep(dstp, basep, kbp):
    blk = _EPR // 8
    spec = pl.BlockSpec((blk, 128), lambda i: (i, 0))
    return pl.pallas_call(
        _prep_body,
        grid=(8,),
        in_specs=[spec] * 3,
        out_specs=[spec, spec],
        out_shape=[jax.ShapeDtypeStruct((_EPR, 128), jnp.int32)] * 2,
    )(dstp, basep, kbp)


def _mul_body(a_ref, b_ref, o_ref):
    o_ref[...] = a_ref[...] * b_ref[...]


def _mul(a, b):
    spec = pl.BlockSpec((_RB, D), lambda i: (i, 0))
    return pl.pallas_call(
        _mul_body,
        grid=(_GRID,),
        in_specs=[spec, spec],
        out_specs=spec,
        out_shape=jax.ShapeDtypeStruct((N, D), jnp.float32),
    )(a, b)


def _mm1_body(x_ref, a0_ref, a1_ref, w_ref, s_ref, st_ref):
    i = pl.program_id(0)
    xx = x_ref[...] + a0_ref[0] + a1_ref[0]
    sv = jnp.dot(xx, w_ref[...], preferred_element_type=jnp.float32)
    s_ref[...] = sv
    ps = jnp.sum(sv, axis=0, keepdims=True)
    pq = jnp.sum(sv * sv, axis=0, keepdims=True)
    blk = jnp.concatenate(
        [ps, pq, jnp.zeros((6, sv.shape[1]), jnp.float32)], axis=0)

    @pl.when(i == 0)
    def _():
        st_ref[...] = jnp.zeros_like(st_ref)

    st_ref[...] += blk


def _mm1(xin, agg, w1):
    return pl.pallas_call(
        _mm1_body,
        grid=(_GRID,),
        in_specs=[
            pl.BlockSpec((_RB, D), lambda i: (i, 0)),
            pl.BlockSpec((1, _RB, D), lambda i: (0, i, 0)),
            pl.BlockSpec((1, _RB, D), lambda i: (1, i, 0)),
            pl.BlockSpec((D, H), lambda i: (0, 0)),
        ],
        out_specs=[
            pl.BlockSpec((_RB, H), lambda i: (i, 0)),
            pl.BlockSpec((8, H), lambda i: (0, 0)),
        ],
        out_shape=[
            jax.ShapeDtypeStruct((N, H), jnp.float32),
            jax.ShapeDtypeStruct((8, H), jnp.float32),
        ],
    )(xin, agg, agg, w1)


def _mm2_body(s_ref, ac_ref, w_ref, h_ref, st_ref):
    i = pl.program_id(0)
    t = jnp.maximum(s_ref[...] * ac_ref[0:1, :] + ac_ref[1:2, :], 0.0)
    hv = jnp.dot(t, w_ref[...], preferred_element_type=jnp.float32)
    h_ref[...] = hv
    ps = jnp.sum(hv, axis=0, keepdims=True)
    pq = jnp.sum(hv * hv, axis=0, keepdims=True)
    blk = jnp.concatenate(
        [ps, pq, jnp.zeros((6, hv.shape[1]), jnp.float32)], axis=0)

    @pl.when(i == 0)
    def _():
        st_ref[...] = jnp.zeros_like(st_ref)

    st_ref[...] += blk


def _mm2(s1, ac1, w2):
    return pl.pallas_call(
        _mm2_body,
        grid=(_GRID,),
        in_specs=[
            pl.BlockSpec((_RB, H), lambda i: (i, 0)),
            pl.BlockSpec((8, H), lambda i: (0, 0)),
            pl.BlockSpec((H, D), lambda i: (0, 0)),
        ],
        out_specs=[
            pl.BlockSpec((_RB, D), lambda i: (i, 0)),
            pl.BlockSpec((8, D), lambda i: (0, 0)),
        ],
        out_shape=[
            jax.ShapeDtypeStruct((N, D), jnp.float32),
            jax.ShapeDtypeStruct((8, D), jnp.float32),
        ],
    )(s1, ac1, w2)


def _bnrelu_body(h_ref, ac_ref, o_ref):
    o_ref[...] = jnp.maximum(h_ref[...] * ac_ref[0:1, :] + ac_ref[1:2, :], 0.0)


def _bnrelu(h, ac):
    return pl.pallas_call(
        _bnrelu_body,
        grid=(_GRID,),
        in_specs=[
            pl.BlockSpec((_RB, D), lambda i: (i, 0)),
            pl.BlockSpec((8, D), lambda i: (0, 0)),
        ],
        out_specs=pl.BlockSpec((_RB, D), lambda i: (i, 0)),
        out_shape=jax.ShapeDtypeStruct((N, D), jnp.float32),
    )(h, ac)


def _bnrelu_mask_body(h_ref, ac_ref, k_ref, o_ref):
    o_ref[...] = jnp.maximum(
        h_ref[...] * ac_ref[0:1, :] + ac_ref[1:2, :], 0.0) * k_ref[...]


def _bnrelu_mask(h, ac, kmat):
    return pl.pallas_call(
        _bnrelu_mask_body,
        grid=(_GRID,),
        in_specs=[
            pl.BlockSpec((_RB, D), lambda i: (i, 0)),
            pl.BlockSpec((8, D), lambda i: (0, 0)),
            pl.BlockSpec((_RB, D), lambda i: (i, 0)),
        ],
        out_specs=pl.BlockSpec((_RB, D), lambda i: (i, 0)),
        out_shape=jax.ShapeDtypeStruct((N, D), jnp.float32),
    )(h, ac, kmat)


def _loss_body(r1_ref, r2_ref, x_ref, m1_ref, m2_ref, o_ref):
    i = pl.program_id(0)

    def nrm(v):
        nn = jnp.sqrt(jnp.sum(v * v, axis=1, keepdims=True))
        return v / jnp.maximum(nn, 1e-12)

    n1 = nrm(r1_ref[...])
    n2 = nrm(r2_ref[...])
    nx = nrm(x_ref[...])
    v1 = jnp.sum(n1 * nx * m1_ref[...], axis=0, keepdims=True)
    v2 = jnp.sum(n2 * nx * m2_ref[...], axis=0, keepdims=True)
    v3 = jnp.sum(n2 * n1, axis=0, keepdims=True)
    blk = jnp.concatenate(
        [v1, v2, v3, jnp.zeros((5, D), jnp.float32)], axis=0)

    @pl.when(i == 0)
    def _():
        o_ref[...] = jnp.zeros_like(o_ref)

    o_ref[...] += blk


def _loss(re1, re2, x, m1mat, m2mat):
    spec = pl.BlockSpec((_RB, D), lambda i: (i, 0))
    return pl.pallas_call(
        _loss_body,
        grid=(_GRID,),
        in_specs=[spec] * 5,
        out_specs=pl.BlockSpec((8, D), lambda i: (0, 0)),
        out_shape=jax.ShapeDtypeStruct((8, D), jnp.float32),
    )(re1, re2, x, m1mat, m2mat)


def _affine(st, g, b):
    m = st[0] / N
    v = st[1] / N - m * m
    a = g / jnp.sqrt(v + 1e-5)
    c = b - m * a
    return jnp.concatenate(
        [a[None], c[None], jnp.zeros((6, a.shape[0]), jnp.float32)], axis=0)


def kernel(x, enc0_w1, enc0_bn_g, enc0_bn_b, enc0_w2, enc0_obn_g, enc0_obn_b,
           enc1_w1, enc1_bn_g, enc1_bn_b, enc1_w2, enc1_obn_g, enc1_obn_b,
           dec_w1, dec_bn_g, dec_bn_b, dec_w2, dec_obn_g, dec_obn_b,
           edge_index, batch):
    del batch

    e0 = edge_index[0]
    e1 = edge_index[1]
    src = jnp.concatenate([e0, e1])
    dst = jnp.concatenate([e1, e0])
    keys = src * N + dst
    iota = lax.iota(jnp.int32, E2)
    sk, order = lax.sort_key_val(keys, iota)
    dup = jnp.concatenate([jnp.zeros((1,), dtype=bool), sk[1:] == sk[:-1]])
    base_s = jnp.logical_not(dup).astype(jnp.int32)
    src_s = sk // N
    dst_s = sk - src_s * N

    (keep1, mv1), (keep2, mv2) = _rng_consts()
    keepb_s = jnp.take(keep1 + 2 * keep2, order)
    srcp = _pad_edges(src_s)
    dst1, dst2 = _prep(_pad_edges(dst_s), _pad_edges(base_s),
                       _pad_edges(keepb_s))

    blocks = (
        (enc0_w1, enc0_bn_g, enc0_bn_b, enc0_w2, enc0_obn_g, enc0_obn_b),
        (enc1_w1, enc1_bn_g, enc1_bn_b, enc1_w2, enc1_obn_g, enc1_obn_b),
        (dec_w1, dec_bn_g, dec_bn_b, dec_w2, dec_obn_g, dec_obn_b),
    )

    def gin_layer(xin, dste, blk):
        w1, g1, b1, w2, og, ob = blk
        agg = _sc_agg(srcp, dste, xin)
        s1, st1 = _mm1(xin, agg, w1)
        ac1 = _affine(st1, g1, b1)
        h, st2 = _mm2(s1, ac1, w2)
        ac2 = _affine(st2, og, ob)
        return h, ac2

    def one_pass(kmat, dste):
        x1 = _mul(x, kmat)
        h0p, ac = gin_layer(x1, dste, blocks[0])
        h0 = _bnrelu(h0p, ac)
        h1p, ac = gin_layer(h0, dste, blocks[1])
        reh = _bnrelu_mask(h1p, ac, kmat)
        h2p, ac = gin_layer(reh, dste, blocks[2])
        return _bnrelu(h2p, ac)

    kmat1 = jnp.broadcast_to((1.0 - mv1)[:, None], (N, D))
    kmat2 = jnp.broadcast_to((1.0 - mv2)[:, None], (N, D))
    re1 = one_pass(kmat1, dst1)
    re2 = one_pass(kmat2, dst2)

    m1mat = jnp.broadcast_to(mv1[:, None], (N, D))
    m2mat = jnp.broadcast_to(mv2[:, None], (N, D))
    V = _loss(re1, re2, x, m1mat, m2mat)
    half = jnp.float32(N // 2)
    l1 = (half - jnp.sum(V[0])) / half
    l2 = (half - jnp.sum(V[1])) / half
    cl = (jnp.float32(N) - jnp.sum(V[2])) / jnp.float32(N)
    return l1 + l2 + 0.1 * cl


# speedup vs baseline: 1.0844x; 1.0844x over previous
import functools

import jax
import jax.numpy as jnp
from jax import lax
from jax.experimental import pallas as pl
from jax.experimental.pallas import tpu as pltpu, tpu_sc as plsc

N = 10000
E = 320000
D = 128
H = 512
E2 = 2 * E

_NC, _NS = 2, 16
_NW = _NC * _NS
_CH = 128
_NCHUNK = 320
_NCH0 = 192
_NCH1 = 128
_EPT = _CH * _NCHUNK
_EP = _NS * _EPT
_EPR = _EP // 128
_NT = 10240
_ZR = _NT // _NS
_IB = 32

_RB = 1000
_GRID = N // _RB


def _rng_consts():
    ka, kb = jax.random.split(jax.random.key(42))
    out = []
    for kk in (ka, kb):
        k1, k2 = jax.random.split(kk)
        keep = jax.random.bernoulli(k1, 0.8, (E2,))
        perm = jax.random.permutation(k2, N)
        mvec = jnp.zeros((N,), dtype=bool).at[perm[: N // 2]].set(True)
        out.append((keep.astype(jnp.int32), mvec.astype(jnp.float32)))
    return out


def _pad_edges(v):
    return jnp.concatenate(
        [v, jnp.zeros((_EP - E2,), v.dtype)]).reshape(_EPR, 128)


def _sc_agg_body(src_ref, dst_ref, tbl_ref, out_ref,
                 idx_s, idx_d, buf0, buf1, agg, gsem0, gsem1):
    c = lax.axis_index("c")
    s = lax.axis_index("s")
    tile_base = s * _NCHUNK + c * _NCH0
    ngroups = jnp.where(c == 0, _NCH0 // _IB, _NCH1 // _IB)

    zv = jnp.zeros((16,), jnp.float32)

    def zrow(i, carry):
        for j in range(8):
            buf0[i, pl.ds(j * 16, 16)] = zv
        return carry

    lax.fori_loop(0, _CH, zrow, 0)
    for t in range(_ZR // _CH):
        pltpu.sync_copy(buf0, agg.at[pl.ds(s * _ZR + t * _CH, _CH)])
    plsc.subcore_barrier()

    def group(g, carry):
        base_row = tile_base + g * _IB
        pltpu.sync_copy(src_ref.at[pl.ds(base_row, _IB)], idx_s)
        pltpu.sync_copy(dst_ref.at[pl.ds(base_row, _IB)], idx_d)
        pltpu.async_copy(tbl_ref.at[idx_s.at[0]], buf0, gsem0)
        pltpu.async_copy(tbl_ref.at[idx_s.at[1]], buf1, gsem1)

        def step(i, cc):
            j0 = 2 * i
            pltpu.make_async_copy(tbl_ref.at[idx_s.at[j0]], buf0, gsem0).wait()
            pltpu.sync_copy(buf0, agg.at[idx_d.at[j0]], add=True)

            @pl.when(j0 + 2 < _IB)
            def _():
                pltpu.async_copy(tbl_ref.at[idx_s.at[j0 + 2]], buf0, gsem0)

            j1 = j0 + 1
            pltpu.make_async_copy(tbl_ref.at[idx_s.at[j1]], buf1, gsem1).wait()
            pltpu.sync_copy(buf1, agg.at[idx_d.at[j1]], add=True)

            @pl.when(j1 + 2 < _IB)
            def _():
                pltpu.async_copy(tbl_ref.at[idx_s.at[j1 + 2]], buf1, gsem1)

            return cc

        lax.fori_loop(0, _IB // 2, step, 0)
        return carry

    lax.fori_loop(0, ngroups, group, 0)
    plsc.subcore_barrier()

    for t in range(_ZR // _CH):
        r0 = s * _ZR + t * _CH
        pltpu.sync_copy(agg.at[pl.ds(r0, _CH)], buf0)
        pltpu.sync_copy(buf0, out_ref.at[c, pl.ds(r0, _CH)])


@functools.cache
def _sc_agg_call():
    return pl.kernel(
        _sc_agg_body,
        out_type=jax.ShapeDtypeStruct((_NC, _NT, D), jnp.float32),
        mesh=plsc.VectorSubcoreMesh(core_axis_name="c", subcore_axis_name="s",
                                    num_cores=_NC, num_subcores=_NS),
        scratch_types=[
            pltpu.VMEM((_IB, _CH), jnp.int32),
            pltpu.VMEM((_IB, _CH), jnp.int32),
            pltpu.VMEM((_CH, D), jnp.float32),
            pltpu.VMEM((_CH, D), jnp.float32),
            pltpu.VMEM_SHARED((_NT, D), jnp.float32),
            pltpu.SemaphoreType.DMA,
            pltpu.SemaphoreType.DMA,
        ],
    )


def _sc_agg(srcp, dstp, table):
    return _sc_agg_call()(srcp, dstp, table)


def _prep_body(d_ref, b_ref, kb_ref, o1_ref, o2_ref):
    d = d_ref[...]
    b = b_ref[...]
    kb = kb_ref[...]
    trash = N + lax.broadcasted_iota(jnp.int32, d.shape, 1)
    o1_ref[...] = jnp.where((b & kb & 1) != 0, d, trash)
    o2_ref[...] = jnp.where((b & (kb >> 1) & 1) != 0, d, trash)


def _prep(dstp, basep, kbp):
    blk = _EPR // 8
    spec = pl.BlockSpec((blk, 128), lambda i: (i, 0))
    return pl.pallas_call(
        _prep_body,
        grid=(8,),
        in_specs=[spec] * 3,
        out_specs=[spec, spec],
        out_shape=[jax.ShapeDtypeStruct((_EPR, 128), jnp.int32)] * 2,
    )(dstp, basep, kbp)


def _mul_body(a_ref, b_ref, o_ref):
    o_ref[...] = a_ref[...] * b_ref[...]


def _mul(a, b):
    spec = pl.BlockSpec((_RB, D), lambda i: (i, 0))
    return pl.pallas_call(
        _mul_body,
        grid=(_GRID,),
        in_specs=[spec, spec],
        out_specs=spec,
        out_shape=jax.ShapeDtypeStruct((N, D), jnp.float32),
    )(a, b)


def _mm1_body(x_ref, a0_ref, a1_ref, w_ref, s_ref, st_ref):
    i = pl.program_id(0)
    xx = x_ref[...] + a0_ref[0] + a1_ref[0]
    sv = jnp.dot(xx, w_ref[...], preferred_element_type=jnp.float32)
    s_ref[...] = sv
    ps = jnp.sum(sv, axis=0, keepdims=True)
    pq = jnp.sum(sv * sv, axis=0, keepdims=True)
    blk = jnp.concatenate(
        [ps, pq, jnp.zeros((6, sv.shape[1]), jnp.float32)], axis=0)

    @pl.when(i == 0)
    def _():
        st_ref[...] = jnp.zeros_like(st_ref)

    st_ref[...] += blk


def _mm1(xin, agg, w1):
    return pl.pallas_call(
        _mm1_body,
        grid=(_GRID,),
        in_specs=[
            pl.BlockSpec((_RB, D), lambda i: (i, 0)),
            pl.BlockSpec((1, _RB, D), lambda i: (0, i, 0)),
            pl.BlockSpec((1, _RB, D), lambda i: (1, i, 0)),
            pl.BlockSpec((D, H), lambda i: (0, 0)),
        ],
        out_specs=[
            pl.BlockSpec((_RB, H), lambda i: (i, 0)),
            pl.BlockSpec((8, H), lambda i: (0, 0)),
        ],
        out_shape=[
            jax.ShapeDtypeStruct((N, H), jnp.float32),
            jax.ShapeDtypeStruct((8, H), jnp.float32),
        ],
    )(xin, agg, agg, w1)


def _mm2_body(s_ref, ac_ref, w_ref, h_ref, st_ref):
    i = pl.program_id(0)
    t = jnp.maximum(s_ref[...] * ac_ref[0:1, :] + ac_ref[1:2, :], 0.0)
    hv = jnp.dot(t, w_ref[...], preferred_element_type=jnp.float32)
    h_ref[...] = hv
    ps = jnp.sum(hv, axis=0, keepdims=True)
    pq = jnp.sum(hv * hv, axis=0, keepdims=True)
    blk = jnp.concatenate(
        [ps, pq, jnp.zeros((6, hv.shape[1]), jnp.float32)], axis=0)

    @pl.when(i == 0)
    def _():
        st_ref[...] = jnp.zeros_like(st_ref)

    st_ref[...] += blk


def _mm2(s1, ac1, w2):
    return pl.pallas_call(
        _mm2_body,
        grid=(_GRID,),
        in_specs=[
            pl.BlockSpec((_RB, H), lambda i: (i, 0)),
            pl.BlockSpec((8, H), lambda i: (0, 0)),
            pl.BlockSpec((H, D), lambda i: (0, 0)),
        ],
        out_specs=[
            pl.BlockSpec((_RB, D), lambda i: (i, 0)),
            pl.BlockSpec((8, D), lambda i: (0, 0)),
        ],
        out_shape=[
            jax.ShapeDtypeStruct((N, D), jnp.float32),
            jax.ShapeDtypeStruct((8, D), jnp.float32),
        ],
    )(s1, ac1, w2)


def _bnrelu_body(h_ref, ac_ref, o_ref):
    o_ref[...] = jnp.maximum(h_ref[...] * ac_ref[0:1, :] + ac_ref[1:2, :], 0.0)


def _bnrelu(h, ac):
    return pl.pallas_call(
        _bnrelu_body,
        grid=(_GRID,),
        in_specs=[
            pl.BlockSpec((_RB, D), lambda i: (i, 0)),
            pl.BlockSpec((8, D), lambda i: (0, 0)),
        ],
        out_specs=pl.BlockSpec((_RB, D), lambda i: (i, 0)),
        out_shape=jax.ShapeDtypeStruct((N, D), jnp.float32),
    )(h, ac)


def _bnrelu_mask_body(h_ref, ac_ref, k_ref, o_ref):
    o_ref[...] = jnp.maximum(
        h_ref[...] * ac_ref[0:1, :] + ac_ref[1:2, :], 0.0) * k_ref[...]


def _bnrelu_mask(h, ac, kmat):
    return pl.pallas_call(
        _bnrelu_mask_body,
        grid=(_GRID,),
        in_specs=[
            pl.BlockSpec((_RB, D), lambda i: (i, 0)),
            pl.BlockSpec((8, D), lambda i: (0, 0)),
            pl.BlockSpec((_RB, D), lambda i: (i, 0)),
        ],
        out_specs=pl.BlockSpec((_RB, D), lambda i: (i, 0)),
        out_shape=jax.ShapeDtypeStruct((N, D), jnp.float32),
    )(h, ac, kmat)


def _loss_body(r1_ref, r2_ref, x_ref, m1_ref, m2_ref, o_ref):
    i = pl.program_id(0)

    def nrm(v):
        nn = jnp.sqrt(jnp.sum(v * v, axis=1, keepdims=True))
        return v / jnp.maximum(nn, 1e-12)

    n1 = nrm(r1_ref[...])
    n2 = nrm(r2_ref[...])
    nx = nrm(x_ref[...])
    v1 = jnp.sum(n1 * nx * m1_ref[...], axis=0, keepdims=True)
    v2 = jnp.sum(n2 * nx * m2_ref[...], axis=0, keepdims=True)
    v3 = jnp.sum(n2 * n1, axis=0, keepdims=True)
    blk = jnp.concatenate(
        [v1, v2, v3, jnp.zeros((5, D), jnp.float32)], axis=0)

    @pl.when(i == 0)
    def _():
        o_ref[...] = jnp.zeros_like(o_ref)

    o_ref[...] += blk


def _loss(re1, re2, x, m1mat, m2mat):
    spec = pl.BlockSpec((_RB, D), lambda i: (i, 0))
    return pl.pallas_call(
        _loss_body,
        grid=(_GRID,),
        in_specs=[spec] * 5,
        out_specs=pl.BlockSpec((8, D), lambda i: (0, 0)),
        out_shape=jax.ShapeDtypeStruct((8, D), jnp.float32),
    )(re1, re2, x, m1mat, m2mat)


def _affine(st, g, b):
    m = st[0] / N
    v = st[1] / N - m * m
    a = g / jnp.sqrt(v + 1e-5)
    c = b - m * a
    return jnp.concatenate(
        [a[None], c[None], jnp.zeros((6, a.shape[0]), jnp.float32)], axis=0)


def kernel(x, enc0_w1, enc0_bn_g, enc0_bn_b, enc0_w2, enc0_obn_g, enc0_obn_b,
           enc1_w1, enc1_bn_g, enc1_bn_b, enc1_w2, enc1_obn_g, enc1_obn_b,
           dec_w1, dec_bn_g, dec_bn_b, dec_w2, dec_obn_g, dec_obn_b,
           edge_index, batch):
    del batch

    e0 = edge_index[0]
    e1 = edge_index[1]
    src = jnp.concatenate([e0, e1])
    dst = jnp.concatenate([e1, e0])
    keys = src * N + dst
    iota = lax.iota(jnp.int32, E2)
    sk, order = lax.sort_key_val(keys, iota)
    dup = jnp.concatenate([jnp.zeros((1,), dtype=bool), sk[1:] == sk[:-1]])
    base_s = jnp.logical_not(dup).astype(jnp.int32)
    src_s = sk // N
    dst_s = sk - src_s * N

    (keep1, mv1), (keep2, mv2) = _rng_consts()
    keepb_s = jnp.take(keep1 + 2 * keep2, order)
    srcp = _pad_edges(src_s)
    dst1, dst2 = _prep(_pad_edges(dst_s), _pad_edges(base_s),
                       _pad_edges(keepb_s))

    blocks = (
        (enc0_w1, enc0_bn_g, enc0_bn_b, enc0_w2, enc0_obn_g, enc0_obn_b),
        (enc1_w1, enc1_bn_g, enc1_bn_b, enc1_w2, enc1_obn_g, enc1_obn_b),
        (dec_w1, dec_bn_g, dec_bn_b, dec_w2, dec_obn_g, dec_obn_b),
    )

    def gin_layer(xin, dste, blk):
        w1, g1, b1, w2, og, ob = blk
        agg = _sc_agg(srcp, dste, xin)
        s1, st1 = _mm1(xin, agg, w1)
        ac1 = _affine(st1, g1, b1)
        h, st2 = _mm2(s1, ac1, w2)
        ac2 = _affine(st2, og, ob)
        return h, ac2

    def one_pass(kmat, dste):
        x1 = _mul(x, kmat)
        h0p, ac = gin_layer(x1, dste, blocks[0])
        h0 = _bnrelu(h0p, ac)
        h1p, ac = gin_layer(h0, dste, blocks[1])
        reh = _bnrelu_mask(h1p, ac, kmat)
        h2p, ac = gin_layer(reh, dste, blocks[2])
        return _bnrelu(h2p, ac)

    kmat1 = jnp.broadcast_to((1.0 - mv1)[:, None], (N, D))
    kmat2 = jnp.broadcast_to((1.0 - mv2)[:, None], (N, D))
    re1 = one_pass(kmat1, dst1)
    re2 = one_pass(kmat2, dst2)

    m1mat = jnp.broadcast_to(mv1[:, None], (N, D))
    m2mat = jnp.broadcast_to(mv2[:, None], (N, D))
    V = _loss(re1, re2, x, m1mat, m2mat)
    half = jnp.float32(N // 2)
    l1 = (half - jnp.sum(V[0])) / half
    l2 = (half - jnp.sum(V[1])) / half
    cl = (jnp.float32(N) - jnp.sum(V[2])) / jnp.float32(N)
    return l1 + l2 + 0.1 * cl
